# baseline (speedup 1.0000x reference)
import functools

import jax
import jax.numpy as jnp
from jax import lax
from jax.experimental import pallas as pl
from jax.experimental.pallas import tpu as pltpu
from jax.experimental.pallas import tpu_sc as plsc

_ROWS, _N = 64, 32768
_L = 16
_UNROLL = 8
_STEPS = _N // (_L * _UNROLL)
_PURE = 20
_ITERS = 48


def _vec(v):
    return jnp.full((_L,), v, jnp.float32)


def _make_sdiv(red_v):
    def _sdiv(a, b):
        red_v[...] = _vec(a) / _vec(b)
        return red_v[...][0]
    return _sdiv


def _lane_sum(vec):
    acc = vec[0]
    for j in range(1, _L):
        acc = acc + vec[j]
    return acc


def _lane_max(vec):
    acc = vec[0]
    for j in range(1, _L):
        acc = jnp.maximum(acc, vec[j])
    return acc


def _row_sparsemax(row_v, red_v):
    _sdiv = _make_sdiv(red_v)
    def max_body(i, acc):
        base = i * _L * _UNROLL
        for u in range(_UNROLL):
            acc = jnp.maximum(acc, row_v[pl.ds(base + u * _L, _L)])
        return acc
    acc = lax.fori_loop(0, _STEPS, max_body, _vec(-jnp.inf))
    m = _lane_max(acc)

    def stats(t):
        tv = _vec(t)
        def body(i, carry):
            cnt, sm = carry
            base = i * _L * _UNROLL
            for u in range(_UNROLL):
                v = row_v[pl.ds(base + u * _L, _L)]
                gt = v > tv
                cnt = cnt + jnp.where(gt, 1.0, 0.0)
                sm = sm + jnp.where(gt, v, 0.0)
            return cnt, sm
        cnt, sm = lax.fori_loop(0, _STEPS, body, (_vec(0.0), _vec(0.0)))
        return _lane_sum(cnt), _lane_sum(sm)

    lo = m - 1.0
    hi = m
    k, s = stats(lo)
    k2 = jnp.float32(1.0)
    s2 = m

    def live(i, lo, hi, k, s, k2, s2):
        nb = jnp.maximum(_sdiv(s - 1.0, k), _sdiv(s2 - 1.0, k2))
        f_lo = s - k * lo - 1.0
        eps_f = 1e-6 * jnp.maximum(1.0, jnp.maximum(s, 0.0 - s))
        return jnp.logical_and(nb > lo, f_lo > eps_f)

    def body(i, carry):
        lo, hi, k, s, k2, s2, pt = carry

        def step(carry):
            lo, hi, k, s, k2, s2, pt = carry
            nb = jnp.maximum(_sdiv(s - 1.0, k), _sdiv(s2 - 1.0, k2))
            mid = 0.5 * (lo + hi)
            t = jnp.where(i < _PURE, jnp.where(nb > lo, nb, mid),
                          jnp.maximum(nb, mid))
            t = jnp.clip(t, lo, hi)
            t = jnp.where(t == pt, mid, t)
            kt, st = stats(t)
            ft = st - kt * t - 1.0
            good = ft >= 0.0
            lo = jnp.where(good, t, lo)
            hi = jnp.where(good, hi, t)
            k = jnp.where(good, kt, k)
            s = jnp.where(good, st, s)
            k2 = jnp.where(good, k2, kt)
            s2 = jnp.where(good, s2, st)
            return lo, hi, k, s, k2, s2, t

        return lax.cond(live(i, lo, hi, k, s, k2, s2),
                        step, lambda c: c, carry)

    lo, hi, k, s, k2, s2, _ = lax.fori_loop(
        0, _ITERS, body, (lo, hi, k, s, k2, s2, hi + 1.0))
    tau = _vec(jnp.maximum(_sdiv(s - 1.0, k), _sdiv(s2 - 1.0, k2)))

    def out_body(i, _):
        base = i * _L * _UNROLL
        for u in range(_UNROLL):
            sl = pl.ds(base + u * _L, _L)
            row_v[sl] = jnp.maximum(row_v[sl] - tau, 0.0)
        return 0
    lax.fori_loop(0, _STEPS, out_body, 0)


def _make_sc_kernel():
    info = plsc.get_sparse_core_info()
    nc, ns = info.num_cores, info.num_subcores
    nw = nc * ns
    rows_per_w = _ROWS // nw
    mesh = plsc.VectorSubcoreMesh(core_axis_name="c", subcore_axis_name="s")

    @functools.partial(
        pl.kernel, mesh=mesh,
        out_type=jax.ShapeDtypeStruct((_ROWS, _N), jnp.float32),
        scratch_types=[pltpu.VMEM((_N,), jnp.float32),
                       pltpu.VMEM((_L,), jnp.float32)],
    )
    def k(x_hbm, out_hbm, row_v, red_v):
        wid = lax.axis_index("s") * nc + lax.axis_index("c")
        for r in range(rows_per_w):
            row = wid * rows_per_w + r
            pltpu.sync_copy(x_hbm.at[row], row_v)
            _row_sparsemax(row_v, red_v)
            pltpu.sync_copy(row_v, out_hbm.at[row])

    return k


_sc_kernel = _make_sc_kernel()


@jax.jit
def kernel(x):
    return _sc_kernel(x)
